# baseline (device time: 11347 ns/iter reference)
import jax
import jax.numpy as jnp
from jax import lax
from jax.experimental import pallas as pl
from jax.experimental.pallas import tpu as pltpu

K = 4
SCALE = 6.0 / 127.0
INV_SCALE = 127.0 / 6.0


def kernel(partial, resid, gamma):
    m, d = resid.shape
    rows = m // K
    gamma2 = gamma.reshape(1, d)

    def body(partial_ref, resid_ref, gamma_ref, out_ref,
             p_vmem, r_vmem, g_vmem, o_vmem, send_buf, recv_buf,
             load_sems, out_sems, g_sem, send_sems, recv_sems):
        my_x = lax.axis_index("x")
        my_y = lax.axis_index("y")
        my_z = lax.axis_index("z")
        xpartner = (1 - my_x, my_y, my_z)

        loads_p, loads_r = [], []
        for k in range(K):
            sl = slice(k * rows, (k + 1) * rows)
            cp = pltpu.make_async_copy(
                partial_ref.at[0, sl, :], p_vmem.at[k], load_sems.at[k])
            cp.start()
            loads_p.append(cp)
            cr = pltpu.make_async_copy(
                resid_ref.at[sl, :], r_vmem.at[k], load_sems.at[K + k])
            cr.start()
            loads_r.append(cr)
        cg = pltpu.make_async_copy(gamma_ref, g_vmem, g_sem)
        cg.start()

        for k in range(K):
            loads_p[k].wait()
            send_buf[k] = jnp.clip(
                jnp.round(p_vmem[k] * INV_SCALE), -127.0, 127.0
            ).astype(jnp.int8)

        barrier_sem = pltpu.get_barrier_semaphore()
        pl.semaphore_signal(
            barrier_sem, inc=1,
            device_id=xpartner, device_id_type=pl.DeviceIdType.MESH,
        )
        pl.semaphore_wait(barrier_sem, 1)

        rdmas = []
        for k in range(K):
            rdma = pltpu.make_async_remote_copy(
                src_ref=send_buf.at[k], dst_ref=recv_buf.at[k],
                send_sem=send_sems.at[k], recv_sem=recv_sems.at[k],
                device_id=xpartner, device_id_type=pl.DeviceIdType.MESH,
            )
            rdma.start()
            rdmas.append(rdma)

        cg.wait()
        outs = []
        for k in range(K):
            rdmas[k].wait_recv()
            loads_r[k].wait()
            y = (p_vmem[k]
                 + recv_buf[k].astype(jnp.float32) * SCALE
                 + r_vmem[k])
            ms = jnp.mean(y * y, axis=-1, keepdims=True)
            o_vmem[k] = y * lax.rsqrt(ms + 1e-6) * g_vmem[...]
            co = pltpu.make_async_copy(
                o_vmem.at[k],
                out_ref.at[slice(k * rows, (k + 1) * rows), :],
                out_sems.at[k])
            co.start()
            outs.append(co)

        for k in range(K):
            outs[k].wait()
            rdmas[k].wait_send()

    return pl.pallas_call(
        body,
        out_shape=jax.ShapeDtypeStruct((m, d), jnp.float32),
        in_specs=[
            pl.BlockSpec(memory_space=pl.ANY),
            pl.BlockSpec(memory_space=pl.ANY),
            pl.BlockSpec(memory_space=pl.ANY),
        ],
        out_specs=pl.BlockSpec(memory_space=pl.ANY),
        scratch_shapes=[
            pltpu.VMEM((K, rows, d), jnp.float32),
            pltpu.VMEM((K, rows, d), jnp.float32),
            pltpu.VMEM((1, d), jnp.float32),
            pltpu.VMEM((K, rows, d), jnp.float32),
            pltpu.VMEM((K, rows, d), jnp.int8),
            pltpu.VMEM((K, rows, d), jnp.int8),
            pltpu.SemaphoreType.DMA((2 * K,)),
            pltpu.SemaphoreType.DMA((K,)),
            pltpu.SemaphoreType.DMA,
            pltpu.SemaphoreType.DMA((K,)),
            pltpu.SemaphoreType.DMA((K,)),
        ],
        compiler_params=pltpu.CompilerParams(collective_id=0),
    )(partial, resid, gamma2)


# device time: 11074 ns/iter; 1.0247x vs baseline; 1.0247x over previous
import jax
import jax.numpy as jnp
from jax import lax
from jax.experimental import pallas as pl
from jax.experimental.pallas import tpu as pltpu

K = 4


def kernel(partial, resid, gamma):
    m, d = resid.shape
    rows = m // K
    gamma2 = gamma.reshape(1, d)

    def body(partial_ref, resid_ref, gamma_ref, out_ref,
             send_buf, recv_buf, send_sems, recv_sems):
        my_x = lax.axis_index("x")
        my_y = lax.axis_index("y")
        my_z = lax.axis_index("z")
        xpartner = (1 - my_x, my_y, my_z)

        for k in range(K):
            sl = slice(k * rows, (k + 1) * rows)
            send_buf[k] = jnp.clip(
                jnp.round(partial_ref[0, sl, :] * (127.0 / 6.0)),
                -127.0, 127.0).astype(jnp.int8)

        barrier_sem = pltpu.get_barrier_semaphore()
        pl.semaphore_signal(
            barrier_sem, inc=1,
            device_id=xpartner, device_id_type=pl.DeviceIdType.MESH,
        )
        pl.semaphore_wait(barrier_sem, 1)

        rdmas = []
        for k in range(K):
            rdma = pltpu.make_async_remote_copy(
                src_ref=send_buf.at[k], dst_ref=recv_buf.at[k],
                send_sem=send_sems.at[k], recv_sem=recv_sems.at[k],
                device_id=xpartner, device_id_type=pl.DeviceIdType.MESH,
            )
            rdma.start()
            rdmas.append(rdma)

        for k in range(K):
            sl = slice(k * rows, (k + 1) * rows)
            rdmas[k].wait_recv()
            y = (partial_ref[0, sl, :]
                 + recv_buf[k].astype(jnp.float32) * (6.0 / 127.0)
                 + resid_ref[sl, :])
            ms = jnp.mean(y * y, axis=-1, keepdims=True)
            out_ref[sl, :] = y * lax.rsqrt(ms + 1e-6) * gamma_ref[...]

        for k in range(K):
            rdmas[k].wait_send()

    return pl.pallas_call(
        body,
        out_shape=jax.ShapeDtypeStruct((m, d), jnp.float32),
        in_specs=[
            pl.BlockSpec(memory_space=pltpu.VMEM),
            pl.BlockSpec(memory_space=pltpu.VMEM),
            pl.BlockSpec(memory_space=pltpu.VMEM),
        ],
        out_specs=pl.BlockSpec(memory_space=pltpu.VMEM),
        scratch_shapes=[
            pltpu.VMEM((K, rows, d), jnp.int8),
            pltpu.VMEM((K, rows, d), jnp.int8),
            pltpu.SemaphoreType.DMA((K,)),
            pltpu.SemaphoreType.DMA((K,)),
        ],
        compiler_params=pltpu.CompilerParams(collective_id=0),
    )(partial, resid, gamma2)
